# Initial kernel scaffold; baseline (speedup 1.0000x reference)
import jax
import jax.numpy as jnp
from jax.experimental import pallas as pl


def kernel(x, c1_w, c1_b, c2_w, c2_b, d1_w1, d1_b1, d1_w2, d1_b2, d2_w1, d2_b1, d2_w2, d2_b2, d3_w1, d3_b1, d3_w2, d3_b2, u1_w1, u1_b1, u1_w2, u1_b2, u2_w1, u2_b1, u2_w2, u2_b2, u3_w1, u3_b1, u3_w2, u3_b2, c3_w, c3_b):
    raise NotImplementedError("write your pallas kernel here")


# speedup vs baseline: 2.6826x; 2.6826x over previous
import jax
import jax.numpy as jnp
from jax.experimental import pallas as pl
from jax.experimental.pallas import tpu as pltpu

_BF = jnp.bfloat16
_F32 = jnp.float32


def _leaky(x):
    return jnp.where(x >= 0, x, 0.1 * x)


def _pool_lane(xb, Hi, Wi, C):
    h = xb[:, :C] + xb[:, C:]
    v = h.reshape(Hi, Wi // 2, C)
    v = v.reshape(Hi // 2, 2, Wi // 2, C).sum(axis=1)
    return v * 0.25


def _up2_store(x3, ups):
    xm = jnp.concatenate([x3[:1], x3[:-1]], axis=0)
    xp = jnp.concatenate([x3[1:], x3[-1:]], axis=0)
    ve = 0.25 * xm + 0.75 * x3
    vo = 0.75 * x3 + 0.25 * xp
    for v, r in ((ve, 0), (vo, 1)):
        hm = jnp.concatenate([v[:, :1], v[:, :-1]], axis=1)
        hp = jnp.concatenate([v[:, 1:], v[:, -1:]], axis=1)
        ups[r::2, 0::2, :] = (0.25 * hm + 0.75 * v).astype(ups.dtype)
        ups[r::2, 1::2, :] = (0.75 * v + 0.25 * hp).astype(ups.dtype)


def _patch_write(xc, xv, K, off, Cg):
    Ho, Wo, Cw = xv.shape
    p = (K - 1) // 2
    for dx in range(K):
        lo = max(0, p - dx)
        hi = min(Wo, Wo + p - dx)
        s0 = lo + dx - p
        xc[p:p + Ho, lo:hi, dx * Cg + off:dx * Cg + off + Cw] = \
            xv[:, s0:s0 + (hi - lo), :]


def _conv_dots(xc, w_ref, Ho, Wo, K):
    KC = w_ref.shape[1]
    acc = None
    for dy in range(K):
        lhs = xc[dy:dy + Ho].reshape(Ho * Wo, KC)
        d = jnp.dot(lhs, w_ref[dy], preferred_element_type=_F32)
        acc = d if acc is None else acc + d
    return acc


def _flat_patch_write(xf, xflat, b, L, Wo, K, off, Cg):
    Cw = xflat.shape[1]
    p = (K - 1) // 2
    wo_col = jax.lax.broadcasted_iota(jnp.int32, (L, Cw), 0) % Wo
    for dy in range(K):
        for dx in range(K):
            t = dy * K + dx
            delta = (dy - p) * Wo + (dx - p)
            lo = max(0, -delta)
            hi = min(L, L - delta)
            src = xflat[lo + delta:hi + delta]
            parts = []
            if lo:
                parts.append(jnp.zeros((lo, Cw), xflat.dtype))
            parts.append(src)
            if L - hi:
                parts.append(jnp.zeros((L - hi, Cw), xflat.dtype))
            slab = jnp.concatenate(parts, axis=0) if len(parts) > 1 else src
            valid = ((wo_col + dx - p) >= 0) & ((wo_col + dx - p) < Wo)
            slab = jnp.where(valid, slab, jnp.zeros_like(slab))
            xf[b * L:(b + 1) * L, t * Cg + off:t * Cg + off + Cw] = slab


def _make_block_body(B, Hi, Wi, Ho, Wo, K1, K2, Cin, Cmid, Cout, Cskip,
                     pre, last_act):
    Cg2 = Cmid + Cskip

    def body(*refs):
        it = iter(refs)
        x_ref = next(it)
        skp_ref = next(it) if Cskip else None
        w1 = next(it)
        b1 = next(it)
        w2 = next(it)
        b2 = next(it)
        o_ref = next(it)
        xc1 = next(it)
        xc2 = next(it)
        ups = next(it) if pre == "up" else None

        xc1[...] = jnp.zeros_like(xc1)
        xc2[...] = jnp.zeros_like(xc2)

        for b in range(B):
            if pre == "pool":
                x3 = _pool_lane(x_ref[b].astype(_F32), Hi, Wi, Cin)
                xv = x3.reshape(Ho, Wo, Cin).astype(_BF)
            elif pre == "up":
                x3 = x_ref[b].astype(_F32).reshape(Hi, Wi, Cin)
                _up2_store(x3, ups)
                xv = ups[...].astype(_BF)
            else:
                xv = x_ref[b].astype(_BF).reshape(Ho, Wo, Cin)
            _patch_write(xc1, xv, K1, 0, Cin)
            y = _leaky(_conv_dots(xc1, w1, Ho, Wo, K1) + b1[...])
            _patch_write(xc2, y.astype(_BF).reshape(Ho, Wo, Cmid),
                         K2, 0, Cg2)
            if Cskip:
                _patch_write(xc2, skp_ref[b].reshape(Ho, Wo, Cskip),
                             K2, Cmid, Cg2)
            acc = _conv_dots(xc2, w2, Ho, Wo, K2) + b2[...]
            if last_act:
                acc = _leaky(acc)
            o_ref[b] = acc.astype(o_ref.dtype)

    return body


def _make_flat_block_body(B, Hi, Wi, Ho, Wo, K, Cin, Cmid, Cout, Cskip, pre):
    Cg2 = Cmid + Cskip
    L = Ho * Wo

    def body(*refs):
        it = iter(refs)
        x_ref = next(it)
        skp_ref = next(it) if Cskip else None
        w1 = next(it)
        b1 = next(it)
        w2 = next(it)
        b2 = next(it)
        o_ref = next(it)
        xf1 = next(it)
        xf2 = next(it)

        for b in range(B):
            if pre == "pool":
                x3 = _pool_lane(x_ref[b].astype(_F32), Hi, Wi, Cin)
                xflat = x3.reshape(L, Cin).astype(_BF)
            else:
                x3 = x_ref[b].astype(_F32).reshape(Hi, Wi, Cin)
                xm = jnp.concatenate([x3[:1], x3[:-1]], axis=0)
                xp = jnp.concatenate([x3[1:], x3[-1:]], axis=0)
                ve = 0.25 * xm + 0.75 * x3
                vo = 0.75 * x3 + 0.25 * xp
                y2 = jnp.stack([ve, vo], axis=1).reshape(Ho, Wi, Cin)
                hm = jnp.concatenate([y2[:, :1], y2[:, :-1]], axis=1)
                hp = jnp.concatenate([y2[:, 1:], y2[:, -1:]], axis=1)
                he = 0.25 * hm + 0.75 * y2
                ho_ = 0.75 * y2 + 0.25 * hp
                up = jnp.stack([he, ho_], axis=2).reshape(Ho, Wo, Cin)
                xflat = up.reshape(L, Cin).astype(_BF)
            _flat_patch_write(xf1, xflat, b, L, Wo, K, 0, Cin)

        y = _leaky(jnp.dot(xf1[...], w1[...],
                           preferred_element_type=_F32) + b1[...])
        yb = y.astype(_BF)
        for b in range(B):
            _flat_patch_write(xf2, yb[b * L:(b + 1) * L], b, L, Wo, K,
                              0, Cg2)
            if Cskip:
                _flat_patch_write(xf2, skp_ref[b], b, L, Wo, K, Cmid, Cg2)

        acc = jnp.dot(xf2[...], w2[...], preferred_element_type=_F32)
        acc = _leaky(acc + b2[...])
        o_ref[...] = acc.reshape(B, L, Cout).astype(o_ref.dtype)

    return body


def _block(x2, *, w1, b1, w2, b2, K1, K2, in_hw, out_hw, pre=None,
           skip=None, B=1, out_dtype=_BF, flat=False):
    N = x2.shape[0]
    Hi, Wi = in_hw
    Ho, Wo = out_hw
    Cin = w1.shape[2]
    Cmid = w1.shape[-1]
    Cout = w2.shape[-1]
    Cskip = skip.shape[2] if skip is not None else 0
    Cg2 = Cmid + Cskip

    b1r = b1.reshape(1, Cmid).astype(_F32)
    b2r = b2.reshape(1, Cout).astype(_F32)

    if flat:
        w1r = w1.astype(_BF).reshape(K1 * K1 * Cin, Cmid)
        w2r = w2.astype(_BF).reshape(K2 * K2 * Cg2, Cout)
        body = _make_flat_block_body(B, Hi, Wi, Ho, Wo, K1, Cin, Cmid,
                                     Cout, Cskip, pre)
        scratch = [pltpu.VMEM((B * Ho * Wo, K1 * K1 * Cin), _BF),
                   pltpu.VMEM((B * Ho * Wo, K2 * K2 * Cg2), _BF)]
        wspecs = [pl.BlockSpec(w1r.shape, lambda n: (0, 0)),
                  pl.BlockSpec(b1r.shape, lambda n: (0, 0)),
                  pl.BlockSpec(w2r.shape, lambda n: (0, 0)),
                  pl.BlockSpec(b2r.shape, lambda n: (0, 0))]
    else:
        w1r = w1.astype(_BF).reshape(K1, K1 * Cin, Cmid)
        w2r = w2.astype(_BF).reshape(K2, K2 * Cg2, Cout)
        body = _make_block_body(B, Hi, Wi, Ho, Wo, K1, K2, Cin, Cmid,
                                Cout, Cskip, pre, True)
        scratch = [pltpu.VMEM((Ho + K1 - 1, Wo, K1 * Cin), _BF),
                   pltpu.VMEM((Ho + K2 - 1, Wo, K2 * Cg2), _BF)]
        if pre == "up":
            scratch.append(pltpu.VMEM((Ho, Wo, Cin), _F32))
        wspecs = [pl.BlockSpec(w1r.shape, lambda n: (0, 0, 0)),
                  pl.BlockSpec(b1r.shape, lambda n: (0, 0)),
                  pl.BlockSpec(w2r.shape, lambda n: (0, 0, 0)),
                  pl.BlockSpec(b2r.shape, lambda n: (0, 0))]

    if pre == "pool":
        xin = x2.reshape(N, Hi * Wi // 2, 2 * Cin)
    else:
        xin = x2
    Lx = xin.shape[1]
    Cx = xin.shape[2]

    inputs = [xin]
    in_specs = [pl.BlockSpec((B, Lx, Cx), lambda n: (n, 0, 0))]
    if Cskip:
        inputs.append(skip)
        in_specs.append(pl.BlockSpec((B, Ho * Wo, Cskip),
                                     lambda n: (n, 0, 0)))
    inputs += [w1r, b1r, w2r, b2r]
    in_specs += wspecs

    return pl.pallas_call(
        body,
        out_shape=jax.ShapeDtypeStruct((N, Ho * Wo, Cout), out_dtype),
        grid_spec=pltpu.PrefetchScalarGridSpec(
            num_scalar_prefetch=0,
            grid=(N // B,),
            in_specs=in_specs,
            out_specs=pl.BlockSpec((B, Ho * Wo, Cout), lambda n: (n, 0, 0)),
            scratch_shapes=scratch),
        compiler_params=pltpu.CompilerParams(
            dimension_semantics=("parallel",)),
    )(*inputs)


def _final_conv(x2, w, b, hw):
    N, L, Cin = x2.shape
    Ho, Wo = hw
    K = w.shape[0]
    Cout = w.shape[-1]
    wr = w.astype(_BF).reshape(K, K * Cin, Cout)
    br = b.reshape(1, Cout).astype(_F32)

    def body(x_ref, w_ref, b_ref, o_ref, xc):
        xc[...] = jnp.zeros_like(xc)
        xv = x_ref[0].astype(_BF).reshape(Ho, Wo, Cin)
        _patch_write(xc, xv, K, 0, Cin)
        acc = _conv_dots(xc, w_ref, Ho, Wo, K) + b_ref[...]
        o_ref[0] = acc.astype(o_ref.dtype)

    return pl.pallas_call(
        body,
        out_shape=jax.ShapeDtypeStruct((N, L, Cout), _F32),
        grid_spec=pltpu.PrefetchScalarGridSpec(
            num_scalar_prefetch=0,
            grid=(N,),
            in_specs=[pl.BlockSpec((1, L, Cin), lambda n: (n, 0, 0)),
                      pl.BlockSpec(wr.shape, lambda n: (0, 0, 0)),
                      pl.BlockSpec(br.shape, lambda n: (0, 0))],
            out_specs=pl.BlockSpec((1, L, Cout), lambda n: (n, 0, 0)),
            scratch_shapes=[pltpu.VMEM((Ho + K - 1, Wo, K * Cin), _BF)]),
        compiler_params=pltpu.CompilerParams(
            dimension_semantics=("parallel",)),
    )(x2, wr, br)


def kernel(x, c1_w, c1_b, c2_w, c2_b, d1_w1, d1_b1, d1_w2, d1_b2,
           d2_w1, d2_b1, d2_w2, d2_b2, d3_w1, d3_b1, d3_w2, d3_b2,
           u1_w1, u1_b1, u1_w2, u1_b2, u2_w1, u2_b1, u2_w2, u2_b2,
           u3_w1, u3_b1, u3_w2, u3_b2, c3_w, c3_b):
    N, H, W, Cin0 = x.shape
    x2 = x.reshape(N, H * W, Cin0)

    s1 = _block(x2, w1=c1_w, b1=c1_b, w2=c2_w, b2=c2_b, K1=7, K2=7,
                in_hw=(H, W), out_hw=(H, W), B=1)
    s2 = _block(s1, w1=d1_w1, b1=d1_b1, w2=d1_w2, b2=d1_b2, K1=5, K2=5,
                in_hw=(H, W), out_hw=(H // 2, W // 2), pre="pool", B=2)
    s3 = _block(s2, w1=d2_w1, b1=d2_b1, w2=d2_w2, b2=d2_b2, K1=3, K2=3,
                in_hw=(H // 2, W // 2), out_hw=(H // 4, W // 4),
                pre="pool", B=4)
    x4 = _block(s3, w1=d3_w1, b1=d3_b1, w2=d3_w2, b2=d3_b2, K1=3, K2=3,
                in_hw=(H // 4, W // 4), out_hw=(H // 8, W // 8),
                pre="pool", B=8, flat=True)
    x5 = _block(x4, w1=u1_w1, b1=u1_b1, w2=u1_w2, b2=u1_b2, K1=3, K2=3,
                in_hw=(H // 8, W // 8), out_hw=(H // 4, W // 4),
                pre="up", skip=s3, B=4, flat=True)
    x6 = _block(x5, w1=u2_w1, b1=u2_b1, w2=u2_w2, b2=u2_b2, K1=3, K2=3,
                in_hw=(H // 4, W // 4), out_hw=(H // 2, W // 2),
                pre="up", skip=s2, B=4)
    x1 = _block(x6, w1=u3_w1, b1=u3_b1, w2=u3_w2, b2=u3_b2, K1=3, K2=3,
                in_hw=(H // 2, W // 2), out_hw=(H, W),
                pre="up", skip=s1, B=1, out_dtype=_F32)
    out = _final_conv(x1, c3_w, c3_b, (H, W))

    return out.reshape(N, H, W, -1), x1.reshape(N, H, W, -1)


# speedup vs baseline: 2.8329x; 1.0560x over previous
import jax
import jax.numpy as jnp
from jax.experimental import pallas as pl
from jax.experimental.pallas import tpu as pltpu

_BF = jnp.bfloat16
_F32 = jnp.float32


def _leaky(x):
    return jnp.where(x >= 0, x, 0.1 * x)


def _pool_lane(xb, Hi, Wi, C, psc):
    psc[...] = xb
    h = psc[0::2, :] + psc[1::2, :]
    v = h.reshape(Hi, Wi // 2, C)
    v = v.reshape(Hi // 2, 2, Wi // 2, C).sum(axis=1)
    return v * 0.25


def _up2_store(x3, ups):
    xm = jnp.concatenate([x3[:1], x3[:-1]], axis=0)
    xp = jnp.concatenate([x3[1:], x3[-1:]], axis=0)
    ve = 0.25 * xm + 0.75 * x3
    vo = 0.75 * x3 + 0.25 * xp
    for v, r in ((ve, 0), (vo, 1)):
        hm = jnp.concatenate([v[:, :1], v[:, :-1]], axis=1)
        hp = jnp.concatenate([v[:, 1:], v[:, -1:]], axis=1)
        ups[r::2, 0::2, :] = (0.25 * hm + 0.75 * v).astype(ups.dtype)
        ups[r::2, 1::2, :] = (0.75 * v + 0.25 * hp).astype(ups.dtype)


def _patch_write(xc, xv, K, off, Cg):
    Ho, Wo, Cw = xv.shape
    p = (K - 1) // 2
    for dx in range(K):
        lo = max(0, p - dx)
        hi = min(Wo, Wo + p - dx)
        s0 = lo + dx - p
        xc[p:p + Ho, lo:hi, dx * Cg + off:dx * Cg + off + Cw] = \
            xv[:, s0:s0 + (hi - lo), :]


def _conv_dots(xc, w_ref, Ho, Wo, K):
    KC = w_ref.shape[1]
    acc = None
    for dy in range(K):
        lhs = xc[dy:dy + Ho].reshape(Ho * Wo, KC)
        d = jnp.dot(lhs, w_ref[dy], preferred_element_type=_F32)
        acc = d if acc is None else acc + d
    return acc


def _flat_patch_write(xf, xflat, b, L, Wo, K, off, Cg):
    Cw = xflat.shape[1]
    p = (K - 1) // 2
    wo_col = jax.lax.broadcasted_iota(jnp.int32, (L, Cw), 0) % Wo
    for dy in range(K):
        for dx in range(K):
            t = dy * K + dx
            delta = (dy - p) * Wo + (dx - p)
            lo = max(0, -delta)
            hi = min(L, L - delta)
            src = xflat[lo + delta:hi + delta]
            parts = []
            if lo:
                parts.append(jnp.zeros((lo, Cw), xflat.dtype))
            parts.append(src)
            if L - hi:
                parts.append(jnp.zeros((L - hi, Cw), xflat.dtype))
            slab = jnp.concatenate(parts, axis=0) if len(parts) > 1 else src
            valid = ((wo_col + dx - p) >= 0) & ((wo_col + dx - p) < Wo)
            slab = jnp.where(valid, slab, jnp.zeros_like(slab))
            xf[b * L:(b + 1) * L, t * Cg + off:t * Cg + off + Cw] = slab


def _make_block_body(B, Hi, Wi, Ho, Wo, K1, K2, Cin, Cmid, Cout, Cskip,
                     pre, last_act):
    Cg2 = Cmid + Cskip

    def body(*refs):
        it = iter(refs)
        x_ref = next(it)
        skp_ref = next(it) if Cskip else None
        w1 = next(it)
        b1 = next(it)
        w2 = next(it)
        b2 = next(it)
        o_ref = next(it)
        xc1 = next(it)
        xc2 = next(it)
        ups = next(it) if pre == "up" else None
        psc = next(it) if pre == "pool" else None

        xc1[...] = jnp.zeros_like(xc1)
        xc2[...] = jnp.zeros_like(xc2)

        for b in range(B):
            if pre == "pool":
                x3 = _pool_lane(x_ref[b].astype(_F32), Hi, Wi, Cin, psc)
                xv = x3.reshape(Ho, Wo, Cin).astype(_BF)
            elif pre == "up":
                x3 = x_ref[b].astype(_F32).reshape(Hi, Wi, Cin)
                _up2_store(x3, ups)
                xv = ups[...].astype(_BF)
            else:
                xv = x_ref[b].astype(_BF).reshape(Ho, Wo, Cin)
            _patch_write(xc1, xv, K1, 0, Cin)
            y = _leaky(_conv_dots(xc1, w1, Ho, Wo, K1) + b1[...])
            _patch_write(xc2, y.astype(_BF).reshape(Ho, Wo, Cmid),
                         K2, 0, Cg2)
            if Cskip:
                _patch_write(xc2, skp_ref[b].reshape(Ho, Wo, Cskip),
                             K2, Cmid, Cg2)
            acc = _conv_dots(xc2, w2, Ho, Wo, K2) + b2[...]
            if last_act:
                acc = _leaky(acc)
            o_ref[b] = acc.astype(o_ref.dtype)

    return body


def _make_flat_block_body(B, Hi, Wi, Ho, Wo, K, Cin, Cmid, Cout, Cskip, pre):
    Cg2 = Cmid + Cskip
    L = Ho * Wo

    def body(*refs):
        it = iter(refs)
        x_ref = next(it)
        skp_ref = next(it) if Cskip else None
        w1 = next(it)
        b1 = next(it)
        w2 = next(it)
        b2 = next(it)
        o_ref = next(it)
        xf1 = next(it)
        xf2 = next(it)
        psc = next(it) if pre == "pool" else None

        for b in range(B):
            if pre == "pool":
                x3 = _pool_lane(x_ref[b].astype(_F32), Hi, Wi, Cin, psc)
                xflat = x3.reshape(L, Cin).astype(_BF)
            else:
                x3 = x_ref[b].astype(_F32).reshape(Hi, Wi, Cin)
                xm = jnp.concatenate([x3[:1], x3[:-1]], axis=0)
                xp = jnp.concatenate([x3[1:], x3[-1:]], axis=0)
                ve = 0.25 * xm + 0.75 * x3
                vo = 0.75 * x3 + 0.25 * xp
                y2 = jnp.stack([ve, vo], axis=1).reshape(Ho, Wi, Cin)
                hm = jnp.concatenate([y2[:, :1], y2[:, :-1]], axis=1)
                hp = jnp.concatenate([y2[:, 1:], y2[:, -1:]], axis=1)
                he = 0.25 * hm + 0.75 * y2
                ho_ = 0.75 * y2 + 0.25 * hp
                up = jnp.stack([he, ho_], axis=2).reshape(Ho, Wo, Cin)
                xflat = up.reshape(L, Cin).astype(_BF)
            _flat_patch_write(xf1, xflat, b, L, Wo, K, 0, Cin)

        y = _leaky(jnp.dot(xf1[...], w1[...],
                           preferred_element_type=_F32) + b1[...])
        yb = y.astype(_BF)
        for b in range(B):
            _flat_patch_write(xf2, yb[b * L:(b + 1) * L], b, L, Wo, K,
                              0, Cg2)
            if Cskip:
                _flat_patch_write(xf2, skp_ref[b], b, L, Wo, K, Cmid, Cg2)

        acc = jnp.dot(xf2[...], w2[...], preferred_element_type=_F32)
        acc = _leaky(acc + b2[...])
        o_ref[...] = acc.reshape(B, L, Cout).astype(o_ref.dtype)

    return body


def _block(x2, *, w1, b1, w2, b2, K1, K2, in_hw, out_hw, pre=None,
           skip=None, B=1, out_dtype=_BF, flat=False):
    N = x2.shape[0]
    Hi, Wi = in_hw
    Ho, Wo = out_hw
    Cin = w1.shape[2]
    Cmid = w1.shape[-1]
    Cout = w2.shape[-1]
    Cskip = skip.shape[2] if skip is not None else 0
    Cg2 = Cmid + Cskip

    b1r = b1.reshape(1, Cmid).astype(_F32)
    b2r = b2.reshape(1, Cout).astype(_F32)

    if flat:
        w1r = w1.astype(_BF).reshape(K1 * K1 * Cin, Cmid)
        w2r = w2.astype(_BF).reshape(K2 * K2 * Cg2, Cout)
        body = _make_flat_block_body(B, Hi, Wi, Ho, Wo, K1, Cin, Cmid,
                                     Cout, Cskip, pre)
        scratch = [pltpu.VMEM((B * Ho * Wo, K1 * K1 * Cin), _BF),
                   pltpu.VMEM((B * Ho * Wo, K2 * K2 * Cg2), _BF)]
        if pre == "pool":
            scratch.append(pltpu.VMEM((Hi * Wi, Cin), _F32))
        wspecs = [pl.BlockSpec(w1r.shape, lambda n: (0, 0)),
                  pl.BlockSpec(b1r.shape, lambda n: (0, 0)),
                  pl.BlockSpec(w2r.shape, lambda n: (0, 0)),
                  pl.BlockSpec(b2r.shape, lambda n: (0, 0))]
    else:
        w1r = w1.astype(_BF).reshape(K1, K1 * Cin, Cmid)
        w2r = w2.astype(_BF).reshape(K2, K2 * Cg2, Cout)
        body = _make_block_body(B, Hi, Wi, Ho, Wo, K1, K2, Cin, Cmid,
                                Cout, Cskip, pre, True)
        scratch = [pltpu.VMEM((Ho + K1 - 1, Wo, K1 * Cin), _BF),
                   pltpu.VMEM((Ho + K2 - 1, Wo, K2 * Cg2), _BF)]
        if pre == "up":
            scratch.append(pltpu.VMEM((Ho, Wo, Cin), _F32))
        if pre == "pool":
            scratch.append(pltpu.VMEM((Hi * Wi, Cin), _F32))
        wspecs = [pl.BlockSpec(w1r.shape, lambda n: (0, 0, 0)),
                  pl.BlockSpec(b1r.shape, lambda n: (0, 0)),
                  pl.BlockSpec(w2r.shape, lambda n: (0, 0, 0)),
                  pl.BlockSpec(b2r.shape, lambda n: (0, 0))]

    xin = x2
    Lx = xin.shape[1]
    Cx = xin.shape[2]

    inputs = [xin]
    in_specs = [pl.BlockSpec((B, Lx, Cx), lambda n: (n, 0, 0))]
    if Cskip:
        inputs.append(skip)
        in_specs.append(pl.BlockSpec((B, Ho * Wo, Cskip),
                                     lambda n: (n, 0, 0)))
    inputs += [w1r, b1r, w2r, b2r]
    in_specs += wspecs

    return pl.pallas_call(
        body,
        out_shape=jax.ShapeDtypeStruct((N, Ho * Wo, Cout), out_dtype),
        grid_spec=pltpu.PrefetchScalarGridSpec(
            num_scalar_prefetch=0,
            grid=(N // B,),
            in_specs=in_specs,
            out_specs=pl.BlockSpec((B, Ho * Wo, Cout), lambda n: (n, 0, 0)),
            scratch_shapes=scratch),
        compiler_params=pltpu.CompilerParams(
            dimension_semantics=("parallel",)),
    )(*inputs)


def _final_conv(x2, w, b, hw):
    N, L, Cin = x2.shape
    Ho, Wo = hw
    K = w.shape[0]
    Cout = w.shape[-1]
    wr = w.astype(_BF).reshape(K, K * Cin, Cout)
    br = b.reshape(1, Cout).astype(_F32)

    def body(x_ref, w_ref, b_ref, o_ref, xc):
        xc[...] = jnp.zeros_like(xc)
        xv = x_ref[0].astype(_BF).reshape(Ho, Wo, Cin)
        _patch_write(xc, xv, K, 0, Cin)
        acc = _conv_dots(xc, w_ref, Ho, Wo, K) + b_ref[...]
        o_ref[0] = acc.astype(o_ref.dtype)

    return pl.pallas_call(
        body,
        out_shape=jax.ShapeDtypeStruct((N, L, Cout), _F32),
        grid_spec=pltpu.PrefetchScalarGridSpec(
            num_scalar_prefetch=0,
            grid=(N,),
            in_specs=[pl.BlockSpec((1, L, Cin), lambda n: (n, 0, 0)),
                      pl.BlockSpec(wr.shape, lambda n: (0, 0, 0)),
                      pl.BlockSpec(br.shape, lambda n: (0, 0))],
            out_specs=pl.BlockSpec((1, L, Cout), lambda n: (n, 0, 0)),
            scratch_shapes=[pltpu.VMEM((Ho + K - 1, Wo, K * Cin), _BF)]),
        compiler_params=pltpu.CompilerParams(
            dimension_semantics=("parallel",)),
    )(x2, wr, br)


def kernel(x, c1_w, c1_b, c2_w, c2_b, d1_w1, d1_b1, d1_w2, d1_b2,
           d2_w1, d2_b1, d2_w2, d2_b2, d3_w1, d3_b1, d3_w2, d3_b2,
           u1_w1, u1_b1, u1_w2, u1_b2, u2_w1, u2_b1, u2_w2, u2_b2,
           u3_w1, u3_b1, u3_w2, u3_b2, c3_w, c3_b):
    N, H, W, Cin0 = x.shape
    x2 = x.reshape(N, H * W, Cin0)

    s1 = _block(x2, w1=c1_w, b1=c1_b, w2=c2_w, b2=c2_b, K1=7, K2=7,
                in_hw=(H, W), out_hw=(H, W), B=1)
    s2 = _block(s1, w1=d1_w1, b1=d1_b1, w2=d1_w2, b2=d1_b2, K1=5, K2=5,
                in_hw=(H, W), out_hw=(H // 2, W // 2), pre="pool", B=2)
    s3 = _block(s2, w1=d2_w1, b1=d2_b1, w2=d2_w2, b2=d2_b2, K1=3, K2=3,
                in_hw=(H // 2, W // 2), out_hw=(H // 4, W // 4),
                pre="pool", B=4)
    x4 = _block(s3, w1=d3_w1, b1=d3_b1, w2=d3_w2, b2=d3_b2, K1=3, K2=3,
                in_hw=(H // 4, W // 4), out_hw=(H // 8, W // 8),
                pre="pool", B=8, flat=True)
    x5 = _block(x4, w1=u1_w1, b1=u1_b1, w2=u1_w2, b2=u1_b2, K1=3, K2=3,
                in_hw=(H // 8, W // 8), out_hw=(H // 4, W // 4),
                pre="up", skip=s3, B=4, flat=True)
    x6 = _block(x5, w1=u2_w1, b1=u2_b1, w2=u2_w2, b2=u2_b2, K1=3, K2=3,
                in_hw=(H // 4, W // 4), out_hw=(H // 2, W // 2),
                pre="up", skip=s2, B=4)
    x1 = _block(x6, w1=u3_w1, b1=u3_b1, w2=u3_w2, b2=u3_b2, K1=3, K2=3,
                in_hw=(H // 2, W // 2), out_hw=(H, W),
                pre="up", skip=s1, B=1, out_dtype=_F32)
    out = _final_conv(x1, c3_w, c3_b, (H, W))

    return out.reshape(N, H, W, -1), x1.reshape(N, H, W, -1)


# speedup vs baseline: 3.3259x; 1.1740x over previous
import jax
import jax.numpy as jnp
from jax.experimental import pallas as pl
from jax.experimental.pallas import tpu as pltpu

_BF = jnp.bfloat16
_F32 = jnp.float32


def _leaky(x):
    return jnp.where(x >= 0, x, 0.1 * x)


def _pool_lane(xb, Hi, Wi, C, psc):
    psc[...] = xb
    h = psc[0::2, :] + psc[1::2, :]
    v = h.reshape(Hi, Wi // 2, C)
    v = v.reshape(Hi // 2, 2, Wi // 2, C).sum(axis=1)
    return v * 0.25


def _up2_store(x3, ups):
    xm = jnp.concatenate([x3[:1], x3[:-1]], axis=0)
    xp = jnp.concatenate([x3[1:], x3[-1:]], axis=0)
    ve = 0.25 * xm + 0.75 * x3
    vo = 0.75 * x3 + 0.25 * xp
    for v, r in ((ve, 0), (vo, 1)):
        hm = jnp.concatenate([v[:, :1], v[:, :-1]], axis=1)
        hp = jnp.concatenate([v[:, 1:], v[:, -1:]], axis=1)
        ups[r::2, 0::2, :] = (0.25 * hm + 0.75 * v).astype(ups.dtype)
        ups[r::2, 1::2, :] = (0.75 * v + 0.25 * hp).astype(ups.dtype)


def _patch_write(xc, xv, K, off, Cg):
    Ho, Wo, Cw = xv.shape
    p = (K - 1) // 2
    for dx in range(K):
        lo = max(0, p - dx)
        hi = min(Wo, Wo + p - dx)
        s0 = lo + dx - p
        xc[p:p + Ho, lo:hi, dx * Cg + off:dx * Cg + off + Cw] = \
            xv[:, s0:s0 + (hi - lo), :]


def _conv_dots(xc, w_ref, Ho, Wo, K):
    KC = w_ref.shape[1]
    acc = None
    for dy in range(K):
        lhs = xc[dy:dy + Ho].reshape(Ho * Wo, KC)
        d = jnp.dot(lhs, w_ref[dy], preferred_element_type=_F32)
        acc = d if acc is None else acc + d
    return acc


def _flat_patch_write(xf, xflat, b, L, Wo, K, off, Cg):
    Cw = xflat.shape[1]
    p = (K - 1) // 2
    wo_col = jax.lax.broadcasted_iota(jnp.int32, (L, Cw), 0) % Wo
    for dy in range(K):
        for dx in range(K):
            t = dy * K + dx
            delta = (dy - p) * Wo + (dx - p)
            lo = max(0, -delta)
            hi = min(L, L - delta)
            src = xflat[lo + delta:hi + delta]
            parts = []
            if lo:
                parts.append(jnp.zeros((lo, Cw), xflat.dtype))
            parts.append(src)
            if L - hi:
                parts.append(jnp.zeros((L - hi, Cw), xflat.dtype))
            slab = jnp.concatenate(parts, axis=0) if len(parts) > 1 else src
            valid = ((wo_col + dx - p) >= 0) & ((wo_col + dx - p) < Wo)
            slab = jnp.where(valid, slab, jnp.zeros_like(slab))
            xf[b * L:(b + 1) * L, t * Cg + off:t * Cg + off + Cw] = slab


def _make_block_body(B, Hi, Wi, Ho, Wo, K1, K2, Cin, Cmid, Cout, Cskip,
                     pre, last_act):
    Cg2 = Cmid + Cskip

    def body(*refs):
        it = iter(refs)
        x_ref = next(it)
        skp_ref = next(it) if Cskip else None
        w1 = next(it)
        b1 = next(it)
        w2 = next(it)
        b2 = next(it)
        o_ref = next(it)
        xc1 = next(it)
        xc2 = next(it)
        ups = next(it) if pre == "up" else None
        psc = next(it) if pre == "pool" else None

        xc1[...] = jnp.zeros_like(xc1)
        xc2[...] = jnp.zeros_like(xc2)

        for b in range(B):
            if pre == "pool":
                x3 = _pool_lane(x_ref[b].astype(_F32), Hi, Wi, Cin, psc)
                xv = x3.reshape(Ho, Wo, Cin).astype(_BF)
            elif pre == "up":
                x3 = x_ref[b].astype(_F32).reshape(Hi, Wi, Cin)
                _up2_store(x3, ups)
                xv = ups[...].astype(_BF)
            else:
                xv = x_ref[b].astype(_BF)
            _patch_write(xc1, xv, K1, 0, Cin)
            y = _leaky(_conv_dots(xc1, w1, Ho, Wo, K1) + b1[...])
            _patch_write(xc2, y.astype(_BF).reshape(Ho, Wo, Cmid),
                         K2, 0, Cg2)
            if Cskip:
                _patch_write(xc2, skp_ref[b].reshape(Ho, Wo, Cskip),
                             K2, Cmid, Cg2)
            acc = _conv_dots(xc2, w2, Ho, Wo, K2) + b2[...]
            if last_act:
                acc = _leaky(acc)
            o_ref[b] = acc.astype(o_ref.dtype)

    return body


def _make_flat_block_body(B, Hi, Wi, Ho, Wo, K, Cin, Cmid, Cout, Cskip, pre):
    Cg2 = Cmid + Cskip
    L = Ho * Wo

    def body(*refs):
        it = iter(refs)
        x_ref = next(it)
        skp_ref = next(it) if Cskip else None
        w1 = next(it)
        b1 = next(it)
        w2 = next(it)
        b2 = next(it)
        o_ref = next(it)
        xf1 = next(it)
        xf2 = next(it)
        psc = next(it) if pre == "pool" else None

        for b in range(B):
            if pre == "pool":
                x3 = _pool_lane(x_ref[b].astype(_F32), Hi, Wi, Cin, psc)
                xflat = x3.reshape(L, Cin).astype(_BF)
            else:
                x3 = x_ref[b].astype(_F32).reshape(Hi, Wi, Cin)
                xm = jnp.concatenate([x3[:1], x3[:-1]], axis=0)
                xp = jnp.concatenate([x3[1:], x3[-1:]], axis=0)
                ve = 0.25 * xm + 0.75 * x3
                vo = 0.75 * x3 + 0.25 * xp
                y2 = jnp.stack([ve, vo], axis=1).reshape(Ho, Wi, Cin)
                hm = jnp.concatenate([y2[:, :1], y2[:, :-1]], axis=1)
                hp = jnp.concatenate([y2[:, 1:], y2[:, -1:]], axis=1)
                he = 0.25 * hm + 0.75 * y2
                ho_ = 0.75 * y2 + 0.25 * hp
                up = jnp.stack([he, ho_], axis=2).reshape(Ho, Wo, Cin)
                xflat = up.reshape(L, Cin).astype(_BF)
            _flat_patch_write(xf1, xflat, b, L, Wo, K, 0, Cin)

        y = _leaky(jnp.dot(xf1[...], w1[...],
                           preferred_element_type=_F32) + b1[...])
        yb = y.astype(_BF)
        for b in range(B):
            _flat_patch_write(xf2, yb[b * L:(b + 1) * L], b, L, Wo, K,
                              0, Cg2)
            if Cskip:
                _flat_patch_write(xf2, skp_ref[b], b, L, Wo, K, Cmid, Cg2)

        acc = jnp.dot(xf2[...], w2[...], preferred_element_type=_F32)
        acc = _leaky(acc + b2[...])
        o_ref[...] = acc.reshape(B, L, Cout).astype(o_ref.dtype)

    return body


def _block(x2, *, w1, b1, w2, b2, K1, K2, in_hw, out_hw, pre=None,
           skip=None, B=1, out_dtype=_BF, flat=False):
    N = x2.shape[0]
    Hi, Wi = in_hw
    Ho, Wo = out_hw
    Cin = w1.shape[2]
    Cmid = w1.shape[-1]
    Cout = w2.shape[-1]
    Cskip = skip.shape[2] if skip is not None else 0
    Cg2 = Cmid + Cskip

    b1r = b1.reshape(1, Cmid).astype(_F32)
    b2r = b2.reshape(1, Cout).astype(_F32)

    if flat:
        w1r = w1.astype(_BF).reshape(K1 * K1 * Cin, Cmid)
        w2r = w2.astype(_BF).reshape(K2 * K2 * Cg2, Cout)
        body = _make_flat_block_body(B, Hi, Wi, Ho, Wo, K1, Cin, Cmid,
                                     Cout, Cskip, pre)
        scratch = [pltpu.VMEM((B * Ho * Wo, K1 * K1 * Cin), _BF),
                   pltpu.VMEM((B * Ho * Wo, K2 * K2 * Cg2), _BF)]
        if pre == "pool":
            scratch.append(pltpu.VMEM((Hi * Wi, Cin), _F32))
        wspecs = [pl.BlockSpec(w1r.shape, lambda n: (0, 0)),
                  pl.BlockSpec(b1r.shape, lambda n: (0, 0)),
                  pl.BlockSpec(w2r.shape, lambda n: (0, 0)),
                  pl.BlockSpec(b2r.shape, lambda n: (0, 0))]
    else:
        w1r = w1.astype(_BF).reshape(K1, K1 * Cin, Cmid)
        w2r = w2.astype(_BF).reshape(K2, K2 * Cg2, Cout)
        body = _make_block_body(B, Hi, Wi, Ho, Wo, K1, K2, Cin, Cmid,
                                Cout, Cskip, pre, True)
        scratch = [pltpu.VMEM((Ho + K1 - 1, Wo, K1 * Cin), _BF),
                   pltpu.VMEM((Ho + K2 - 1, Wo, K2 * Cg2), _BF)]
        if pre == "up":
            scratch.append(pltpu.VMEM((Ho, Wo, Cin), _F32))
        if pre == "pool":
            scratch.append(pltpu.VMEM((Hi * Wi, Cin), _F32))
        wspecs = [pl.BlockSpec(w1r.shape, lambda n: (0, 0, 0)),
                  pl.BlockSpec(b1r.shape, lambda n: (0, 0)),
                  pl.BlockSpec(w2r.shape, lambda n: (0, 0, 0)),
                  pl.BlockSpec(b2r.shape, lambda n: (0, 0))]

    inputs = [x2]
    if x2.ndim == 4:
        in_specs = [pl.BlockSpec((B, Hi, Wi, x2.shape[3]),
                                 lambda n: (n, 0, 0, 0))]
    else:
        in_specs = [pl.BlockSpec((B, x2.shape[1], x2.shape[2]),
                                 lambda n: (n, 0, 0))]
    if Cskip:
        inputs.append(skip)
        in_specs.append(pl.BlockSpec((B, Ho * Wo, Cskip),
                                     lambda n: (n, 0, 0)))
    inputs += [w1r, b1r, w2r, b2r]
    in_specs += wspecs

    return pl.pallas_call(
        body,
        out_shape=jax.ShapeDtypeStruct((N, Ho * Wo, Cout), out_dtype),
        grid_spec=pltpu.PrefetchScalarGridSpec(
            num_scalar_prefetch=0,
            grid=(N // B,),
            in_specs=in_specs,
            out_specs=pl.BlockSpec((B, Ho * Wo, Cout), lambda n: (n, 0, 0)),
            scratch_shapes=scratch),
        compiler_params=pltpu.CompilerParams(
            dimension_semantics=("parallel",)),
    )(*inputs)


def _up_final_block(x2, skip, w1, b1, w2, b2, w3, b3, in_hw, out_hw):
    N = x2.shape[0]
    Hi, Wi = in_hw
    Ho, Wo = out_hw
    K = 3
    Cin = w1.shape[2]
    Cmid = w1.shape[-1]
    Cout = w2.shape[-1]
    Cskip = skip.shape[2]
    Cg2 = Cmid + Cskip
    C3 = w3.shape[-1]

    w1r = w1.astype(_BF).reshape(K, K * Cin, Cmid)
    w2r = w2.astype(_BF).reshape(K, K * Cg2, Cout)
    w3r = w3.astype(_BF).reshape(K, K * Cout, C3)
    b1r = b1.reshape(1, Cmid).astype(_F32)
    b2r = b2.reshape(1, Cout).astype(_F32)
    b3r = b3.reshape(1, C3).astype(_F32)

    def body(x_ref, skp_ref, w1_, b1_, w2_, b2_, w3_, b3_,
             x1_ref, o_ref, xc1, xc2, xc3, ups):
        xc1[...] = jnp.zeros_like(xc1)
        xc2[...] = jnp.zeros_like(xc2)
        xc3[...] = jnp.zeros_like(xc3)
        x3 = x_ref[0].astype(_F32).reshape(Hi, Wi, Cin)
        _up2_store(x3, ups)
        _patch_write(xc1, ups[...].astype(_BF), K, 0, Cin)
        y = _leaky(_conv_dots(xc1, w1_, Ho, Wo, K) + b1_[...])
        _patch_write(xc2, y.astype(_BF).reshape(Ho, Wo, Cmid), K, 0, Cg2)
        _patch_write(xc2, skp_ref[0].reshape(Ho, Wo, Cskip), K, Cmid, Cg2)
        x1v = _leaky(_conv_dots(xc2, w2_, Ho, Wo, K) + b2_[...])
        x1_ref[0] = x1v.reshape(Ho, Wo, Cout).astype(x1_ref.dtype)
        _patch_write(xc3, x1v.astype(_BF).reshape(Ho, Wo, Cout), K, 0, Cout)
        acc3 = _conv_dots(xc3, w3_, Ho, Wo, K) + b3_[...]
        o_ref[0] = acc3.reshape(Ho, Wo, C3).astype(o_ref.dtype)

    return pl.pallas_call(
        body,
        out_shape=(jax.ShapeDtypeStruct((N, Ho, Wo, Cout), _F32),
                   jax.ShapeDtypeStruct((N, Ho, Wo, C3), _F32)),
        grid_spec=pltpu.PrefetchScalarGridSpec(
            num_scalar_prefetch=0,
            grid=(N,),
            in_specs=[pl.BlockSpec((1, Hi * Wi, Cin), lambda n: (n, 0, 0)),
                      pl.BlockSpec((1, Ho * Wo, Cskip),
                                   lambda n: (n, 0, 0)),
                      pl.BlockSpec(w1r.shape, lambda n: (0, 0, 0)),
                      pl.BlockSpec(b1r.shape, lambda n: (0, 0)),
                      pl.BlockSpec(w2r.shape, lambda n: (0, 0, 0)),
                      pl.BlockSpec(b2r.shape, lambda n: (0, 0)),
                      pl.BlockSpec(w3r.shape, lambda n: (0, 0, 0)),
                      pl.BlockSpec(b3r.shape, lambda n: (0, 0))],
            out_specs=(pl.BlockSpec((1, Ho, Wo, Cout),
                                    lambda n: (n, 0, 0, 0)),
                       pl.BlockSpec((1, Ho, Wo, C3),
                                    lambda n: (n, 0, 0, 0))),
            scratch_shapes=[pltpu.VMEM((Ho + K - 1, Wo, K * Cin), _BF),
                            pltpu.VMEM((Ho + K - 1, Wo, K * Cg2), _BF),
                            pltpu.VMEM((Ho + K - 1, Wo, K * Cout), _BF),
                            pltpu.VMEM((Ho, Wo, Cin), _F32)]),
        compiler_params=pltpu.CompilerParams(
            dimension_semantics=("parallel",)),
    )(x2, skip, w1r, b1r, w2r, b2r, w3r, b3r)


def kernel(x, c1_w, c1_b, c2_w, c2_b, d1_w1, d1_b1, d1_w2, d1_b2,
           d2_w1, d2_b1, d2_w2, d2_b2, d3_w1, d3_b1, d3_w2, d3_b2,
           u1_w1, u1_b1, u1_w2, u1_b2, u2_w1, u2_b1, u2_w2, u2_b2,
           u3_w1, u3_b1, u3_w2, u3_b2, c3_w, c3_b):
    N, H, W, Cin0 = x.shape

    s1 = _block(x, w1=c1_w, b1=c1_b, w2=c2_w, b2=c2_b, K1=7, K2=7,
                in_hw=(H, W), out_hw=(H, W), B=1)
    s2 = _block(s1, w1=d1_w1, b1=d1_b1, w2=d1_w2, b2=d1_b2, K1=5, K2=5,
                in_hw=(H, W), out_hw=(H // 2, W // 2), pre="pool", B=2)
    s3 = _block(s2, w1=d2_w1, b1=d2_b1, w2=d2_w2, b2=d2_b2, K1=3, K2=3,
                in_hw=(H // 2, W // 2), out_hw=(H // 4, W // 4),
                pre="pool", B=4)
    x4 = _block(s3, w1=d3_w1, b1=d3_b1, w2=d3_w2, b2=d3_b2, K1=3, K2=3,
                in_hw=(H // 4, W // 4), out_hw=(H // 8, W // 8),
                pre="pool", B=8, flat=True)
    x5 = _block(x4, w1=u1_w1, b1=u1_b1, w2=u1_w2, b2=u1_b2, K1=3, K2=3,
                in_hw=(H // 8, W // 8), out_hw=(H // 4, W // 4),
                pre="up", skip=s3, B=4, flat=True)
    x6 = _block(x5, w1=u2_w1, b1=u2_b1, w2=u2_w2, b2=u2_b2, K1=3, K2=3,
                in_hw=(H // 4, W // 4), out_hw=(H // 2, W // 2),
                pre="up", skip=s2, B=4)
    x1, out = _up_final_block(x6, s1, u3_w1, u3_b1, u3_w2, u3_b2,
                              c3_w, c3_b, (H // 2, W // 2), (H, W))

    return out, x1


# speedup vs baseline: 4.0403x; 1.2148x over previous
import jax
import jax.numpy as jnp
from jax.experimental import pallas as pl
from jax.experimental.pallas import tpu as pltpu

_BF = jnp.bfloat16
_F32 = jnp.float32


def _leaky(x):
    return jnp.where(x >= 0, x, 0.1 * x)


def _pool_lane(xb, Hi, Wi, C, psc):
    psc[...] = xb
    h = psc[0::2, :] + psc[1::2, :]
    v = h.reshape(Hi, Wi // 2, C)
    v = v.reshape(Hi // 2, 2, Wi // 2, C).sum(axis=1)
    return v * 0.25


def _up2_store(x3, ups):
    xm = jnp.concatenate([x3[:1], x3[:-1]], axis=0)
    xp = jnp.concatenate([x3[1:], x3[-1:]], axis=0)
    ve = 0.25 * xm + 0.75 * x3
    vo = 0.75 * x3 + 0.25 * xp
    for v, r in ((ve, 0), (vo, 1)):
        hm = jnp.concatenate([v[:, :1], v[:, :-1]], axis=1)
        hp = jnp.concatenate([v[:, 1:], v[:, -1:]], axis=1)
        ups[r::2, 0::2, :] = (0.25 * hm + 0.75 * v).astype(ups.dtype)
        ups[r::2, 1::2, :] = (0.75 * v + 0.25 * hp).astype(ups.dtype)


def _patch_write(xc, xv, K, off, Cg, base=0, r0=0):
    Rh, Wo, Cw = xv.shape
    p = (K - 1) // 2
    for dx in range(K):
        lo = max(0, p - dx)
        hi = min(Wo, Wo + p - dx)
        s0 = lo + dx - p
        c0 = base + dx * Cg + off
        xc[p + r0:p + r0 + Rh, lo:hi, c0:c0 + Cw] = \
            xv[:, s0:s0 + (hi - lo), :]


def _conv_dots(xc, w_ref, Ho, Wo, K):
    KC = w_ref.shape[1]
    acc = None
    for dy in range(K):
        lhs = xc[dy:dy + Ho].reshape(Ho * Wo, KC)
        d = jnp.dot(lhs, w_ref[dy], preferred_element_type=_F32)
        acc = d if acc is None else acc + d
    return acc


def _flat_patch_write(xf, xflat, b, L, Wo, K, off, Cg):
    Cw = xflat.shape[1]
    p = (K - 1) // 2
    wo_col = jax.lax.broadcasted_iota(jnp.int32, (L, Cw), 0) % Wo
    for dy in range(K):
        for dx in range(K):
            t = dy * K + dx
            delta = (dy - p) * Wo + (dx - p)
            lo = max(0, -delta)
            hi = min(L, L - delta)
            src = xflat[lo + delta:hi + delta]
            parts = []
            if lo:
                parts.append(jnp.zeros((lo, Cw), xflat.dtype))
            parts.append(src)
            if L - hi:
                parts.append(jnp.zeros((L - hi, Cw), xflat.dtype))
            slab = jnp.concatenate(parts, axis=0) if len(parts) > 1 else src
            valid = ((wo_col + dx - p) >= 0) & ((wo_col + dx - p) < Wo)
            slab = jnp.where(valid, slab, jnp.zeros_like(slab))
            xf[b * L:(b + 1) * L, t * Cg + off:t * Cg + off + Cw] = slab


def _make_block_body(B, Hi, Wi, Ho, Wo, K1, K2, Cin, Cmid, Cout, Cskip,
                     pre, last_act):
    Cg2 = Cmid + Cskip

    def body(*refs):
        it = iter(refs)
        x_ref = next(it)
        skp_ref = next(it) if Cskip else None
        w1 = next(it)
        b1 = next(it)
        w2 = next(it)
        b2 = next(it)
        o_ref = next(it)
        xc1 = next(it)
        xc2 = next(it)
        ups = next(it) if pre == "up" else None
        psc = next(it) if pre == "pool" else None

        xc1[...] = jnp.zeros_like(xc1)
        xc2[...] = jnp.zeros_like(xc2)

        for b in range(B):
            if pre == "pool":
                x3 = _pool_lane(x_ref[b].astype(_F32), Hi, Wi, Cin, psc)
                xv = x3.reshape(Ho, Wo, Cin).astype(_BF)
            elif pre == "up":
                x3 = x_ref[b].astype(_F32).reshape(Hi, Wi, Cin)
                _up2_store(x3, ups)
                xv = ups[...].astype(_BF)
            else:
                xv = x_ref[b].astype(_BF)
            _patch_write(xc1, xv, K1, 0, Cin)
            y = _leaky(_conv_dots(xc1, w1, Ho, Wo, K1) + b1[...])
            _patch_write(xc2, y.astype(_BF).reshape(Ho, Wo, Cmid),
                         K2, 0, Cg2)
            if Cskip:
                _patch_write(xc2, skp_ref[b].reshape(Ho, Wo, Cskip),
                             K2, Cmid, Cg2)
            acc = _conv_dots(xc2, w2, Ho, Wo, K2) + b2[...]
            if last_act:
                acc = _leaky(acc)
            o_ref[b] = acc.astype(o_ref.dtype)

    return body


def _kron_eye(w, B):
    K, R, C = w.shape
    eye = jnp.eye(B, dtype=w.dtype)
    return (eye[None, :, None, :, None] *
            w[:, None, :, None, :]).reshape(K, B * R, B * C)


def _make_stacked_block_body(B, Hi, Wi, Ho, Wo, K1, K2, Cin, Cmid, Cout,
                             Cskip, pre):
    Cg2 = Cmid + Cskip
    G1 = K1 * Cin
    G2 = K2 * Cg2

    def body(*refs):
        it = iter(refs)
        x_ref = next(it)
        skp_ref = next(it) if Cskip else None
        w1 = next(it)
        b1 = next(it)
        w2 = next(it)
        b2 = next(it)
        o_ref = next(it)
        xc1 = next(it)
        xc2 = next(it)
        ups = next(it) if pre == "up" else None
        psc = next(it) if pre == "pool" else None

        xc1[...] = jnp.zeros_like(xc1)
        xc2[...] = jnp.zeros_like(xc2)

        for b in range(B):
            if pre == "pool":
                x3 = _pool_lane(x_ref[b].astype(_F32), Hi, Wi, Cin, psc)
                xv = x3.reshape(Ho, Wo, Cin).astype(_BF)
            elif pre == "up":
                x3 = x_ref[b].astype(_F32).reshape(Hi, Wi, Cin)
                _up2_store(x3, ups)
                xv = ups[...].astype(_BF)
            else:
                xv = x_ref[b].astype(_BF)
            _patch_write(xc1, xv, K1, 0, Cin, base=b * G1)
            if Cskip:
                _patch_write(xc2, skp_ref[b].reshape(Ho, Wo, Cskip),
                             K2, Cmid, Cg2, base=b * G2)

        R = max(1, 1024 // Wo) if Ho * Wo > 1024 else max(1, 512 // Wo)
        KC1 = B * G1
        KC2 = B * G2
        for r0 in range(0, Ho, R):
            acc = None
            for dy in range(K1):
                lhs = xc1[r0 + dy:r0 + dy + R].reshape(R * Wo, KC1)
                d = jnp.dot(lhs, w1[dy], preferred_element_type=_F32)
                acc = d if acc is None else acc + d
            v = _leaky(acc + b1[...]).astype(_BF)
            for b in range(B):
                _patch_write(xc2,
                             v[:, b * Cmid:(b + 1) * Cmid].reshape(
                                 R, Wo, Cmid),
                             K2, 0, Cg2, base=b * G2, r0=r0)
        for r0 in range(0, Ho, R):
            acc = None
            for dy in range(K2):
                lhs = xc2[r0 + dy:r0 + dy + R].reshape(R * Wo, KC2)
                d = jnp.dot(lhs, w2[dy], preferred_element_type=_F32)
                acc = d if acc is None else acc + d
            v = _leaky(acc + b2[...])
            for b in range(B):
                o_ref[b, r0 * Wo:(r0 + R) * Wo, :] = \
                    v[:, b * Cout:(b + 1) * Cout].astype(o_ref.dtype)

    return body


def _make_flat_block_body(B, Hi, Wi, Ho, Wo, K, Cin, Cmid, Cout, Cskip, pre):
    Cg2 = Cmid + Cskip
    L = Ho * Wo

    def body(*refs):
        it = iter(refs)
        x_ref = next(it)
        skp_ref = next(it) if Cskip else None
        w1 = next(it)
        b1 = next(it)
        w2 = next(it)
        b2 = next(it)
        o_ref = next(it)
        xf1 = next(it)
        xf2 = next(it)
        psc = next(it) if pre == "pool" else None

        for b in range(B):
            if pre == "pool":
                x3 = _pool_lane(x_ref[b].astype(_F32), Hi, Wi, Cin, psc)
                xflat = x3.reshape(L, Cin).astype(_BF)
            else:
                x3 = x_ref[b].astype(_F32).reshape(Hi, Wi, Cin)
                xm = jnp.concatenate([x3[:1], x3[:-1]], axis=0)
                xp = jnp.concatenate([x3[1:], x3[-1:]], axis=0)
                ve = 0.25 * xm + 0.75 * x3
                vo = 0.75 * x3 + 0.25 * xp
                y2 = jnp.stack([ve, vo], axis=1).reshape(Ho, Wi, Cin)
                hm = jnp.concatenate([y2[:, :1], y2[:, :-1]], axis=1)
                hp = jnp.concatenate([y2[:, 1:], y2[:, -1:]], axis=1)
                he = 0.25 * hm + 0.75 * y2
                ho_ = 0.75 * y2 + 0.25 * hp
                up = jnp.stack([he, ho_], axis=2).reshape(Ho, Wo, Cin)
                xflat = up.reshape(L, Cin).astype(_BF)
            _flat_patch_write(xf1, xflat, b, L, Wo, K, 0, Cin)

        y = _leaky(jnp.dot(xf1[...], w1[...],
                           preferred_element_type=_F32) + b1[...])
        yb = y.astype(_BF)
        for b in range(B):
            _flat_patch_write(xf2, yb[b * L:(b + 1) * L], b, L, Wo, K,
                              0, Cg2)
            if Cskip:
                _flat_patch_write(xf2, skp_ref[b], b, L, Wo, K, Cmid, Cg2)

        acc = jnp.dot(xf2[...], w2[...], preferred_element_type=_F32)
        acc = _leaky(acc + b2[...])
        o_ref[...] = acc.reshape(B, L, Cout).astype(o_ref.dtype)

    return body


def _block(x2, *, w1, b1, w2, b2, K1, K2, in_hw, out_hw, pre=None,
           skip=None, B=1, out_dtype=_BF, flat=False, stacked=False):
    N = x2.shape[0]
    Hi, Wi = in_hw
    Ho, Wo = out_hw
    Cin = w1.shape[2]
    Cmid = w1.shape[-1]
    Cout = w2.shape[-1]
    Cskip = skip.shape[2] if skip is not None else 0
    Cg2 = Cmid + Cskip

    b1r = b1.reshape(1, Cmid).astype(_F32)
    b2r = b2.reshape(1, Cout).astype(_F32)

    if flat:
        w1r = w1.astype(_BF).reshape(K1 * K1 * Cin, Cmid)
        w2r = w2.astype(_BF).reshape(K2 * K2 * Cg2, Cout)
        body = _make_flat_block_body(B, Hi, Wi, Ho, Wo, K1, Cin, Cmid,
                                     Cout, Cskip, pre)
        scratch = [pltpu.VMEM((B * Ho * Wo, K1 * K1 * Cin), _BF),
                   pltpu.VMEM((B * Ho * Wo, K2 * K2 * Cg2), _BF)]
        if pre == "pool":
            scratch.append(pltpu.VMEM((Hi * Wi, Cin), _F32))
        wspecs = [pl.BlockSpec(w1r.shape, lambda n: (0, 0)),
                  pl.BlockSpec(b1r.shape, lambda n: (0, 0)),
                  pl.BlockSpec(w2r.shape, lambda n: (0, 0)),
                  pl.BlockSpec(b2r.shape, lambda n: (0, 0))]
    elif stacked:
        w1r = _kron_eye(w1.astype(_BF).reshape(K1, K1 * Cin, Cmid), B)
        w2r = _kron_eye(w2.astype(_BF).reshape(K2, K2 * Cg2, Cout), B)
        b1r = jnp.tile(b1r, (1, B))
        b2r = jnp.tile(b2r, (1, B))
        body = _make_stacked_block_body(B, Hi, Wi, Ho, Wo, K1, K2, Cin,
                                        Cmid, Cout, Cskip, pre)
        scratch = [pltpu.VMEM((Ho + K1 - 1, Wo, B * K1 * Cin), _BF),
                   pltpu.VMEM((Ho + K2 - 1, Wo, B * K2 * Cg2), _BF)]
        if pre == "up":
            scratch.append(pltpu.VMEM((Ho, Wo, Cin), _F32))
        if pre == "pool":
            scratch.append(pltpu.VMEM((Hi * Wi, Cin), _F32))
        wspecs = [pl.BlockSpec(w1r.shape, lambda n: (0, 0, 0)),
                  pl.BlockSpec(b1r.shape, lambda n: (0, 0)),
                  pl.BlockSpec(w2r.shape, lambda n: (0, 0, 0)),
                  pl.BlockSpec(b2r.shape, lambda n: (0, 0))]
    else:
        w1r = w1.astype(_BF).reshape(K1, K1 * Cin, Cmid)
        w2r = w2.astype(_BF).reshape(K2, K2 * Cg2, Cout)
        body = _make_block_body(B, Hi, Wi, Ho, Wo, K1, K2, Cin, Cmid,
                                Cout, Cskip, pre, True)
        scratch = [pltpu.VMEM((Ho + K1 - 1, Wo, K1 * Cin), _BF),
                   pltpu.VMEM((Ho + K2 - 1, Wo, K2 * Cg2), _BF)]
        if pre == "up":
            scratch.append(pltpu.VMEM((Ho, Wo, Cin), _F32))
        if pre == "pool":
            scratch.append(pltpu.VMEM((Hi * Wi, Cin), _F32))
        wspecs = [pl.BlockSpec(w1r.shape, lambda n: (0, 0, 0)),
                  pl.BlockSpec(b1r.shape, lambda n: (0, 0)),
                  pl.BlockSpec(w2r.shape, lambda n: (0, 0, 0)),
                  pl.BlockSpec(b2r.shape, lambda n: (0, 0))]

    inputs = [x2]
    if x2.ndim == 4:
        in_specs = [pl.BlockSpec((B, Hi, Wi, x2.shape[3]),
                                 lambda n: (n, 0, 0, 0))]
    else:
        in_specs = [pl.BlockSpec((B, x2.shape[1], x2.shape[2]),
                                 lambda n: (n, 0, 0))]
    if Cskip:
        inputs.append(skip)
        in_specs.append(pl.BlockSpec((B, Ho * Wo, Cskip),
                                     lambda n: (n, 0, 0)))
    inputs += [w1r, b1r, w2r, b2r]
    in_specs += wspecs

    return pl.pallas_call(
        body,
        out_shape=jax.ShapeDtypeStruct((N, Ho * Wo, Cout), out_dtype),
        grid_spec=pltpu.PrefetchScalarGridSpec(
            num_scalar_prefetch=0,
            grid=(N // B,),
            in_specs=in_specs,
            out_specs=pl.BlockSpec((B, Ho * Wo, Cout), lambda n: (n, 0, 0)),
            scratch_shapes=scratch),
        compiler_params=pltpu.CompilerParams(
            dimension_semantics=("parallel",)),
    )(*inputs)


def _up_final_block(x2, skip, w1, b1, w2, b2, w3, b3, in_hw, out_hw):
    N = x2.shape[0]
    Hi, Wi = in_hw
    Ho, Wo = out_hw
    K = 3
    Cin = w1.shape[2]
    Cmid = w1.shape[-1]
    Cout = w2.shape[-1]
    Cskip = skip.shape[2]
    Cg2 = Cmid + Cskip
    C3 = w3.shape[-1]

    w1r = w1.astype(_BF).reshape(K, K * Cin, Cmid)
    w2r = w2.astype(_BF).reshape(K, K * Cg2, Cout)
    w3r = w3.astype(_BF).reshape(K, K * Cout, C3)
    b1r = b1.reshape(1, Cmid).astype(_F32)
    b2r = b2.reshape(1, Cout).astype(_F32)
    b3r = b3.reshape(1, C3).astype(_F32)

    def body(x_ref, skp_ref, w1_, b1_, w2_, b2_, w3_, b3_,
             x1_ref, o_ref, xc1, xc2, xc3, ups):
        xc1[...] = jnp.zeros_like(xc1)
        xc2[...] = jnp.zeros_like(xc2)
        xc3[...] = jnp.zeros_like(xc3)
        x3 = x_ref[0].astype(_F32).reshape(Hi, Wi, Cin)
        _up2_store(x3, ups)
        _patch_write(xc1, ups[...].astype(_BF), K, 0, Cin)
        y = _leaky(_conv_dots(xc1, w1_, Ho, Wo, K) + b1_[...])
        _patch_write(xc2, y.astype(_BF).reshape(Ho, Wo, Cmid), K, 0, Cg2)
        _patch_write(xc2, skp_ref[0].reshape(Ho, Wo, Cskip), K, Cmid, Cg2)
        x1v = _leaky(_conv_dots(xc2, w2_, Ho, Wo, K) + b2_[...])
        x1_ref[0] = x1v.reshape(Ho, Wo, Cout).astype(x1_ref.dtype)
        _patch_write(xc3, x1v.astype(_BF).reshape(Ho, Wo, Cout), K, 0, Cout)
        acc3 = _conv_dots(xc3, w3_, Ho, Wo, K) + b3_[...]
        o_ref[0] = acc3.reshape(Ho, Wo, C3).astype(o_ref.dtype)

    return pl.pallas_call(
        body,
        out_shape=(jax.ShapeDtypeStruct((N, Ho, Wo, Cout), _F32),
                   jax.ShapeDtypeStruct((N, Ho, Wo, C3), _F32)),
        grid_spec=pltpu.PrefetchScalarGridSpec(
            num_scalar_prefetch=0,
            grid=(N,),
            in_specs=[pl.BlockSpec((1, Hi * Wi, Cin), lambda n: (n, 0, 0)),
                      pl.BlockSpec((1, Ho * Wo, Cskip),
                                   lambda n: (n, 0, 0)),
                      pl.BlockSpec(w1r.shape, lambda n: (0, 0, 0)),
                      pl.BlockSpec(b1r.shape, lambda n: (0, 0)),
                      pl.BlockSpec(w2r.shape, lambda n: (0, 0, 0)),
                      pl.BlockSpec(b2r.shape, lambda n: (0, 0)),
                      pl.BlockSpec(w3r.shape, lambda n: (0, 0, 0)),
                      pl.BlockSpec(b3r.shape, lambda n: (0, 0))],
            out_specs=(pl.BlockSpec((1, Ho, Wo, Cout),
                                    lambda n: (n, 0, 0, 0)),
                       pl.BlockSpec((1, Ho, Wo, C3),
                                    lambda n: (n, 0, 0, 0))),
            scratch_shapes=[pltpu.VMEM((Ho + K - 1, Wo, K * Cin), _BF),
                            pltpu.VMEM((Ho + K - 1, Wo, K * Cg2), _BF),
                            pltpu.VMEM((Ho + K - 1, Wo, K * Cout), _BF),
                            pltpu.VMEM((Ho, Wo, Cin), _F32)]),
        compiler_params=pltpu.CompilerParams(
            dimension_semantics=("parallel",)),
    )(x2, skip, w1r, b1r, w2r, b2r, w3r, b3r)


def kernel(x, c1_w, c1_b, c2_w, c2_b, d1_w1, d1_b1, d1_w2, d1_b2,
           d2_w1, d2_b1, d2_w2, d2_b2, d3_w1, d3_b1, d3_w2, d3_b2,
           u1_w1, u1_b1, u1_w2, u1_b2, u2_w1, u2_b1, u2_w2, u2_b2,
           u3_w1, u3_b1, u3_w2, u3_b2, c3_w, c3_b):
    N, H, W, Cin0 = x.shape

    s1 = _block(x, w1=c1_w, b1=c1_b, w2=c2_w, b2=c2_b, K1=7, K2=7,
                in_hw=(H, W), out_hw=(H, W), B=4, stacked=True)
    s2 = _block(s1, w1=d1_w1, b1=d1_b1, w2=d1_w2, b2=d1_b2, K1=5, K2=5,
                in_hw=(H, W), out_hw=(H // 2, W // 2), pre="pool", B=4,
                stacked=True)
    s3 = _block(s2, w1=d2_w1, b1=d2_b1, w2=d2_w2, b2=d2_b2, K1=3, K2=3,
                in_hw=(H // 2, W // 2), out_hw=(H // 4, W // 4),
                pre="pool", B=4)
    x4 = _block(s3, w1=d3_w1, b1=d3_b1, w2=d3_w2, b2=d3_b2, K1=3, K2=3,
                in_hw=(H // 4, W // 4), out_hw=(H // 8, W // 8),
                pre="pool", B=8, flat=True)
    x5 = _block(x4, w1=u1_w1, b1=u1_b1, w2=u1_w2, b2=u1_b2, K1=3, K2=3,
                in_hw=(H // 8, W // 8), out_hw=(H // 4, W // 4),
                pre="up", skip=s3, B=4, flat=True)
    x6 = _block(x5, w1=u2_w1, b1=u2_b1, w2=u2_w2, b2=u2_b2, K1=3, K2=3,
                in_hw=(H // 4, W // 4), out_hw=(H // 2, W // 2),
                pre="up", skip=s2, B=4, stacked=True)
    x1, out = _up_final_block(x6, s1, u3_w1, u3_b1, u3_w2, u3_b2,
                              c3_w, c3_b, (H // 2, W // 2), (H, W))

    return out, x1


# speedup vs baseline: 4.2372x; 1.0488x over previous
import jax
import jax.numpy as jnp
from jax.experimental import pallas as pl
from jax.experimental.pallas import tpu as pltpu

_BF = jnp.bfloat16
_F32 = jnp.float32


def _leaky(x):
    return jnp.where(x >= 0, x, 0.1 * x)


def _pool_lane(xb, Hi, Wi, C, psc):
    psc[...] = xb
    h = psc[0::2, :] + psc[1::2, :]
    v = h.reshape(Hi, Wi // 2, C)
    v = v.reshape(Hi // 2, 2, Wi // 2, C).sum(axis=1)
    return v * 0.25


def _up2_store(x3, ups):
    xm = jnp.concatenate([x3[:1], x3[:-1]], axis=0)
    xp = jnp.concatenate([x3[1:], x3[-1:]], axis=0)
    ve = 0.25 * xm + 0.75 * x3
    vo = 0.75 * x3 + 0.25 * xp
    for v, r in ((ve, 0), (vo, 1)):
        hm = jnp.concatenate([v[:, :1], v[:, :-1]], axis=1)
        hp = jnp.concatenate([v[:, 1:], v[:, -1:]], axis=1)
        ups[r::2, 0::2, :] = (0.25 * hm + 0.75 * v).astype(ups.dtype)
        ups[r::2, 1::2, :] = (0.75 * v + 0.25 * hp).astype(ups.dtype)


def _patch_write(xc, xv, K, off, Cg, base=0, r0=0):
    Rh, Wo, Cw = xv.shape
    p = (K - 1) // 2
    for dx in range(K):
        lo = max(0, p - dx)
        hi = min(Wo, Wo + p - dx)
        s0 = lo + dx - p
        c0 = base + dx * Cg + off
        xc[p + r0:p + r0 + Rh, lo:hi, c0:c0 + Cw] = \
            xv[:, s0:s0 + (hi - lo), :]


def _conv_dots(xc, w_ref, Ho, Wo, K):
    KC = w_ref.shape[1]
    acc = None
    for dy in range(K):
        lhs = xc[dy:dy + Ho].reshape(Ho * Wo, KC)
        d = jnp.dot(lhs, w_ref[dy], preferred_element_type=_F32)
        acc = d if acc is None else acc + d
    return acc


def _flat_patch_write(xf, xflat, b, L, Wo, K, off, Cg):
    Cw = xflat.shape[1]
    p = (K - 1) // 2
    wo_col = jax.lax.broadcasted_iota(jnp.int32, (L, Cw), 0) % Wo
    for dy in range(K):
        for dx in range(K):
            t = dy * K + dx
            delta = (dy - p) * Wo + (dx - p)
            lo = max(0, -delta)
            hi = min(L, L - delta)
            src = xflat[lo + delta:hi + delta]
            parts = []
            if lo:
                parts.append(jnp.zeros((lo, Cw), xflat.dtype))
            parts.append(src)
            if L - hi:
                parts.append(jnp.zeros((L - hi, Cw), xflat.dtype))
            slab = jnp.concatenate(parts, axis=0) if len(parts) > 1 else src
            valid = ((wo_col + dx - p) >= 0) & ((wo_col + dx - p) < Wo)
            slab = jnp.where(valid, slab, jnp.zeros_like(slab))
            xf[b * L:(b + 1) * L, t * Cg + off:t * Cg + off + Cw] = slab


def _make_block_body(B, Hi, Wi, Ho, Wo, K1, K2, Cin, Cmid, Cout, Cskip,
                     pre, last_act):
    Cg2 = Cmid + Cskip

    def body(*refs):
        it = iter(refs)
        x_ref = next(it)
        skp_ref = next(it) if Cskip else None
        w1 = next(it)
        b1 = next(it)
        w2 = next(it)
        b2 = next(it)
        o_ref = next(it)
        xc1 = next(it)
        xc2 = next(it)
        ups = next(it) if pre == "up" else None
        psc = next(it) if pre == "pool" else None

        xc1[...] = jnp.zeros_like(xc1)
        xc2[...] = jnp.zeros_like(xc2)

        for b in range(B):
            if pre == "pool":
                x3 = _pool_lane(x_ref[b].astype(_F32), Hi, Wi, Cin, psc)
                xv = x3.reshape(Ho, Wo, Cin).astype(_BF)
            elif pre == "up":
                x3 = x_ref[b].astype(_F32).reshape(Hi, Wi, Cin)
                _up2_store(x3, ups)
                xv = ups[...].astype(_BF)
            else:
                xv = x_ref[b].astype(_BF)
            _patch_write(xc1, xv, K1, 0, Cin)
            y = _leaky(_conv_dots(xc1, w1, Ho, Wo, K1) + b1[...])
            _patch_write(xc2, y.astype(_BF).reshape(Ho, Wo, Cmid),
                         K2, 0, Cg2)
            if Cskip:
                _patch_write(xc2, skp_ref[b].reshape(Ho, Wo, Cskip),
                             K2, Cmid, Cg2)
            acc = _conv_dots(xc2, w2, Ho, Wo, K2) + b2[...]
            if last_act:
                acc = _leaky(acc)
            o_ref[b] = acc.astype(o_ref.dtype)

    return body


def _kron_eye(w, B):
    K, R, C = w.shape
    eye = jnp.eye(B, dtype=w.dtype)
    return (eye[None, :, None, :, None] *
            w[:, None, :, None, :]).reshape(K, B * R, B * C)


def _make_stacked_block_body(B, Hi, Wi, Ho, Wo, K1, K2, Cin, Cmid, Cout,
                             Cskip, pre):
    Cg2 = Cmid + Cskip
    G1 = K1 * Cin
    G2 = K2 * Cg2

    def body(*refs):
        it = iter(refs)
        x_ref = next(it)
        skp_ref = next(it) if Cskip else None
        w1 = next(it)
        b1 = next(it)
        w2 = next(it)
        b2 = next(it)
        o_ref = next(it)
        xc1 = next(it)
        xc2 = next(it)
        ups = next(it) if pre == "up" else None
        psc = next(it) if pre == "pool" else None

        xc1[...] = jnp.zeros_like(xc1)
        xc2[...] = jnp.zeros_like(xc2)

        for b in range(B):
            if pre == "pool":
                x3 = _pool_lane(x_ref[b].astype(_F32), Hi, Wi, Cin, psc)
                xv = x3.reshape(Ho, Wo, Cin).astype(_BF)
            elif pre == "up":
                x3 = x_ref[b].astype(_F32).reshape(Hi, Wi, Cin)
                _up2_store(x3, ups)
                xv = ups[...].astype(_BF)
            else:
                xv = x_ref[b].astype(_BF)
            _patch_write(xc1, xv, K1, 0, Cin, base=b * G1)
            if Cskip:
                _patch_write(xc2, skp_ref[b].reshape(Ho, Wo, Cskip),
                             K2, Cmid, Cg2, base=b * G2)

        R = max(1, 1024 // Wo) if Ho * Wo > 1024 else max(1, 512 // Wo)
        KC1 = B * G1
        KC2 = B * G2
        for r0 in range(0, Ho, R):
            acc = None
            for dy in range(K1):
                lhs = xc1[r0 + dy:r0 + dy + R].reshape(R * Wo, KC1)
                d = jnp.dot(lhs, w1[dy], preferred_element_type=_F32)
                acc = d if acc is None else acc + d
            v = _leaky(acc + b1[...]).astype(_BF)
            for b in range(B):
                _patch_write(xc2,
                             v[:, b * Cmid:(b + 1) * Cmid].reshape(
                                 R, Wo, Cmid),
                             K2, 0, Cg2, base=b * G2, r0=r0)
        for r0 in range(0, Ho, R):
            acc = None
            for dy in range(K2):
                lhs = xc2[r0 + dy:r0 + dy + R].reshape(R * Wo, KC2)
                d = jnp.dot(lhs, w2[dy], preferred_element_type=_F32)
                acc = d if acc is None else acc + d
            v = _leaky(acc + b2[...])
            for b in range(B):
                o_ref[b, r0 * Wo:(r0 + R) * Wo, :] = \
                    v[:, b * Cout:(b + 1) * Cout].astype(o_ref.dtype)

    return body


def _make_flat_block_body(B, Hi, Wi, Ho, Wo, K, Cin, Cmid, Cout, Cskip, pre):
    Cg2 = Cmid + Cskip
    L = Ho * Wo

    def body(*refs):
        it = iter(refs)
        x_ref = next(it)
        skp_ref = next(it) if Cskip else None
        w1 = next(it)
        b1 = next(it)
        w2 = next(it)
        b2 = next(it)
        o_ref = next(it)
        xf1 = next(it)
        xf2 = next(it)
        psc = next(it) if pre == "pool" else None

        for b in range(B):
            if pre == "pool":
                x3 = _pool_lane(x_ref[b].astype(_F32), Hi, Wi, Cin, psc)
                xflat = x3.reshape(L, Cin).astype(_BF)
            else:
                x3 = x_ref[b].astype(_F32).reshape(Hi, Wi, Cin)
                xm = jnp.concatenate([x3[:1], x3[:-1]], axis=0)
                xp = jnp.concatenate([x3[1:], x3[-1:]], axis=0)
                ve = 0.25 * xm + 0.75 * x3
                vo = 0.75 * x3 + 0.25 * xp
                y2 = jnp.stack([ve, vo], axis=1).reshape(Ho, Wi, Cin)
                hm = jnp.concatenate([y2[:, :1], y2[:, :-1]], axis=1)
                hp = jnp.concatenate([y2[:, 1:], y2[:, -1:]], axis=1)
                he = 0.25 * hm + 0.75 * y2
                ho_ = 0.75 * y2 + 0.25 * hp
                up = jnp.stack([he, ho_], axis=2).reshape(Ho, Wo, Cin)
                xflat = up.reshape(L, Cin).astype(_BF)
            _flat_patch_write(xf1, xflat, b, L, Wo, K, 0, Cin)

        y = _leaky(jnp.dot(xf1[...], w1[...],
                           preferred_element_type=_F32) + b1[...])
        yb = y.astype(_BF)
        for b in range(B):
            _flat_patch_write(xf2, yb[b * L:(b + 1) * L], b, L, Wo, K,
                              0, Cg2)
            if Cskip:
                _flat_patch_write(xf2, skp_ref[b], b, L, Wo, K, Cmid, Cg2)

        acc = jnp.dot(xf2[...], w2[...], preferred_element_type=_F32)
        acc = _leaky(acc + b2[...])
        o_ref[...] = acc.reshape(B, L, Cout).astype(o_ref.dtype)

    return body


def _block(x2, *, w1, b1, w2, b2, K1, K2, in_hw, out_hw, pre=None,
           skip=None, B=1, out_dtype=_BF, flat=False, stacked=False):
    N = x2.shape[0]
    Hi, Wi = in_hw
    Ho, Wo = out_hw
    Cin = w1.shape[2]
    Cmid = w1.shape[-1]
    Cout = w2.shape[-1]
    Cskip = skip.shape[2] if skip is not None else 0
    Cg2 = Cmid + Cskip

    b1r = b1.reshape(1, Cmid).astype(_F32)
    b2r = b2.reshape(1, Cout).astype(_F32)

    if flat:
        w1r = w1.astype(_BF).reshape(K1 * K1 * Cin, Cmid)
        w2r = w2.astype(_BF).reshape(K2 * K2 * Cg2, Cout)
        body = _make_flat_block_body(B, Hi, Wi, Ho, Wo, K1, Cin, Cmid,
                                     Cout, Cskip, pre)
        scratch = [pltpu.VMEM((B * Ho * Wo, K1 * K1 * Cin), _BF),
                   pltpu.VMEM((B * Ho * Wo, K2 * K2 * Cg2), _BF)]
        if pre == "pool":
            scratch.append(pltpu.VMEM((Hi * Wi, Cin), _F32))
        wspecs = [pl.BlockSpec(w1r.shape, lambda n: (0, 0)),
                  pl.BlockSpec(b1r.shape, lambda n: (0, 0)),
                  pl.BlockSpec(w2r.shape, lambda n: (0, 0)),
                  pl.BlockSpec(b2r.shape, lambda n: (0, 0))]
    elif stacked:
        w1r = _kron_eye(w1.astype(_BF).reshape(K1, K1 * Cin, Cmid), B)
        w2r = _kron_eye(w2.astype(_BF).reshape(K2, K2 * Cg2, Cout), B)
        b1r = jnp.tile(b1r, (1, B))
        b2r = jnp.tile(b2r, (1, B))
        body = _make_stacked_block_body(B, Hi, Wi, Ho, Wo, K1, K2, Cin,
                                        Cmid, Cout, Cskip, pre)
        scratch = [pltpu.VMEM((Ho + K1 - 1, Wo, B * K1 * Cin), _BF),
                   pltpu.VMEM((Ho + K2 - 1, Wo, B * K2 * Cg2), _BF)]
        if pre == "up":
            scratch.append(pltpu.VMEM((Ho, Wo, Cin), _F32))
        if pre == "pool":
            scratch.append(pltpu.VMEM((Hi * Wi, Cin), _F32))
        wspecs = [pl.BlockSpec(w1r.shape, lambda n: (0, 0, 0)),
                  pl.BlockSpec(b1r.shape, lambda n: (0, 0)),
                  pl.BlockSpec(w2r.shape, lambda n: (0, 0, 0)),
                  pl.BlockSpec(b2r.shape, lambda n: (0, 0))]
    else:
        w1r = w1.astype(_BF).reshape(K1, K1 * Cin, Cmid)
        w2r = w2.astype(_BF).reshape(K2, K2 * Cg2, Cout)
        body = _make_block_body(B, Hi, Wi, Ho, Wo, K1, K2, Cin, Cmid,
                                Cout, Cskip, pre, True)
        scratch = [pltpu.VMEM((Ho + K1 - 1, Wo, K1 * Cin), _BF),
                   pltpu.VMEM((Ho + K2 - 1, Wo, K2 * Cg2), _BF)]
        if pre == "up":
            scratch.append(pltpu.VMEM((Ho, Wo, Cin), _F32))
        if pre == "pool":
            scratch.append(pltpu.VMEM((Hi * Wi, Cin), _F32))
        wspecs = [pl.BlockSpec(w1r.shape, lambda n: (0, 0, 0)),
                  pl.BlockSpec(b1r.shape, lambda n: (0, 0)),
                  pl.BlockSpec(w2r.shape, lambda n: (0, 0, 0)),
                  pl.BlockSpec(b2r.shape, lambda n: (0, 0))]

    inputs = [x2]
    if x2.ndim == 4:
        in_specs = [pl.BlockSpec((B, Hi, Wi, x2.shape[3]),
                                 lambda n: (n, 0, 0, 0))]
    else:
        in_specs = [pl.BlockSpec((B, x2.shape[1], x2.shape[2]),
                                 lambda n: (n, 0, 0))]
    if Cskip:
        inputs.append(skip)
        in_specs.append(pl.BlockSpec((B, Ho * Wo, Cskip),
                                     lambda n: (n, 0, 0)))
    inputs += [w1r, b1r, w2r, b2r]
    in_specs += wspecs

    return pl.pallas_call(
        body,
        out_shape=jax.ShapeDtypeStruct((N, Ho * Wo, Cout), out_dtype),
        grid_spec=pltpu.PrefetchScalarGridSpec(
            num_scalar_prefetch=0,
            grid=(N // B,),
            in_specs=in_specs,
            out_specs=pl.BlockSpec((B, Ho * Wo, Cout), lambda n: (n, 0, 0)),
            scratch_shapes=scratch),
        compiler_params=pltpu.CompilerParams(
            dimension_semantics=("parallel",)),
    )(*inputs)


def _up_final_block(x2, skip, w1, b1, w2, b2, w3, b3, in_hw, out_hw):
    N = x2.shape[0]
    Hi, Wi = in_hw
    Ho, Wo = out_hw
    K = 3
    Cin = w1.shape[2]
    Cmid = w1.shape[-1]
    Cout = w2.shape[-1]
    Cskip = skip.shape[2]
    Cg2 = Cmid + Cskip
    C3 = w3.shape[-1]

    w1r = w1.astype(_BF).reshape(K, K * Cin, Cmid)
    w2r = w2.astype(_BF).reshape(K, K * Cg2, Cout)
    w3r = w3.astype(_BF).reshape(K, K * Cout, C3)
    b1r = b1.reshape(1, Cmid).astype(_F32)
    b2r = b2.reshape(1, Cout).astype(_F32)
    b3r = b3.reshape(1, C3).astype(_F32)

    B = 2
    G1 = K * Cin
    G2 = K * Cg2
    G3 = K * Cout
    w1r = _kron_eye(w1r, B)
    w2r = _kron_eye(w2r, B)
    w3r = _kron_eye(w3r, B)
    b1r = jnp.tile(b1r, (1, B))
    b2r = jnp.tile(b2r, (1, B))
    b3r = jnp.tile(b3r, (1, B))

    def body(x_ref, skp_ref, w1_, b1_, w2_, b2_, w3_, b3_,
             x1_ref, o_ref, xc1, xc2, xc3, ups):
        xc1[...] = jnp.zeros_like(xc1)
        xc2[...] = jnp.zeros_like(xc2)
        xc3[...] = jnp.zeros_like(xc3)
        for b in range(B):
            x3 = x_ref[b].astype(_F32).reshape(Hi, Wi, Cin)
            _up2_store(x3, ups)
            _patch_write(xc1, ups[...].astype(_BF), K, 0, Cin, base=b * G1)
            _patch_write(xc2, skp_ref[b].reshape(Ho, Wo, Cskip),
                         K, Cmid, Cg2, base=b * G2)
        R = 16
        for r0 in range(0, Ho, R):
            acc = None
            for dy in range(K):
                lhs = xc1[r0 + dy:r0 + dy + R].reshape(R * Wo, B * G1)
                d = jnp.dot(lhs, w1_[dy], preferred_element_type=_F32)
                acc = d if acc is None else acc + d
            v = _leaky(acc + b1_[...]).astype(_BF)
            for b in range(B):
                _patch_write(xc2,
                             v[:, b * Cmid:(b + 1) * Cmid].reshape(
                                 R, Wo, Cmid),
                             K, 0, Cg2, base=b * G2, r0=r0)
        for r0 in range(0, Ho, R):
            acc = None
            for dy in range(K):
                lhs = xc2[r0 + dy:r0 + dy + R].reshape(R * Wo, B * G2)
                d = jnp.dot(lhs, w2_[dy], preferred_element_type=_F32)
                acc = d if acc is None else acc + d
            v = _leaky(acc + b2_[...])
            vb = v.astype(_BF)
            for b in range(B):
                x1_ref[b, r0:r0 + R] = \
                    v[:, b * Cout:(b + 1) * Cout].reshape(
                        R, Wo, Cout).astype(x1_ref.dtype)
                _patch_write(xc3,
                             vb[:, b * Cout:(b + 1) * Cout].reshape(
                                 R, Wo, Cout),
                             K, 0, Cout, base=b * G3, r0=r0)
        for r0 in range(0, Ho, R):
            acc = None
            for dy in range(K):
                lhs = xc3[r0 + dy:r0 + dy + R].reshape(R * Wo, B * G3)
                d = jnp.dot(lhs, w3_[dy], preferred_element_type=_F32)
                acc = d if acc is None else acc + d
            v = acc + b3_[...]
            for b in range(B):
                o_ref[b, r0:r0 + R] = \
                    v[:, b * C3:(b + 1) * C3].reshape(
                        R, Wo, C3).astype(o_ref.dtype)

    return pl.pallas_call(
        body,
        out_shape=(jax.ShapeDtypeStruct((N, Ho, Wo, Cout), _F32),
                   jax.ShapeDtypeStruct((N, Ho, Wo, C3), _F32)),
        grid_spec=pltpu.PrefetchScalarGridSpec(
            num_scalar_prefetch=0,
            grid=(N // B,),
            in_specs=[pl.BlockSpec((B, Hi * Wi, Cin), lambda n: (n, 0, 0)),
                      pl.BlockSpec((B, Ho * Wo, Cskip),
                                   lambda n: (n, 0, 0)),
                      pl.BlockSpec(w1r.shape, lambda n: (0, 0, 0)),
                      pl.BlockSpec(b1r.shape, lambda n: (0, 0)),
                      pl.BlockSpec(w2r.shape, lambda n: (0, 0, 0)),
                      pl.BlockSpec(b2r.shape, lambda n: (0, 0)),
                      pl.BlockSpec(w3r.shape, lambda n: (0, 0, 0)),
                      pl.BlockSpec(b3r.shape, lambda n: (0, 0))],
            out_specs=(pl.BlockSpec((B, Ho, Wo, Cout),
                                    lambda n: (n, 0, 0, 0)),
                       pl.BlockSpec((B, Ho, Wo, C3),
                                    lambda n: (n, 0, 0, 0))),
            scratch_shapes=[pltpu.VMEM((Ho + K - 1, Wo, B * G1), _BF),
                            pltpu.VMEM((Ho + K - 1, Wo, B * G2), _BF),
                            pltpu.VMEM((Ho + K - 1, Wo, B * G3), _BF),
                            pltpu.VMEM((Ho, Wo, Cin), _F32)]),
        compiler_params=pltpu.CompilerParams(
            dimension_semantics=("parallel",)),
    )(x2, skip, w1r, b1r, w2r, b2r, w3r, b3r)


def kernel(x, c1_w, c1_b, c2_w, c2_b, d1_w1, d1_b1, d1_w2, d1_b2,
           d2_w1, d2_b1, d2_w2, d2_b2, d3_w1, d3_b1, d3_w2, d3_b2,
           u1_w1, u1_b1, u1_w2, u1_b2, u2_w1, u2_b1, u2_w2, u2_b2,
           u3_w1, u3_b1, u3_w2, u3_b2, c3_w, c3_b):
    N, H, W, Cin0 = x.shape

    s1 = _block(x, w1=c1_w, b1=c1_b, w2=c2_w, b2=c2_b, K1=7, K2=7,
                in_hw=(H, W), out_hw=(H, W), B=4, stacked=True)
    s2 = _block(s1, w1=d1_w1, b1=d1_b1, w2=d1_w2, b2=d1_b2, K1=5, K2=5,
                in_hw=(H, W), out_hw=(H // 2, W // 2), pre="pool", B=4,
                stacked=True)
    s3 = _block(s2, w1=d2_w1, b1=d2_b1, w2=d2_w2, b2=d2_b2, K1=3, K2=3,
                in_hw=(H // 2, W // 2), out_hw=(H // 4, W // 4),
                pre="pool", B=4)
    x4 = _block(s3, w1=d3_w1, b1=d3_b1, w2=d3_w2, b2=d3_b2, K1=3, K2=3,
                in_hw=(H // 4, W // 4), out_hw=(H // 8, W // 8),
                pre="pool", B=8, flat=True)
    x5 = _block(x4, w1=u1_w1, b1=u1_b1, w2=u1_w2, b2=u1_b2, K1=3, K2=3,
                in_hw=(H // 8, W // 8), out_hw=(H // 4, W // 4),
                pre="up", skip=s3, B=4, flat=True)
    x6 = _block(x5, w1=u2_w1, b1=u2_b1, w2=u2_w2, b2=u2_b2, K1=3, K2=3,
                in_hw=(H // 4, W // 4), out_hw=(H // 2, W // 2),
                pre="up", skip=s2, B=4, stacked=True)
    x1, out = _up_final_block(x6, s1, u3_w1, u3_b1, u3_w2, u3_b2,
                              c3_w, c3_b, (H // 2, W // 2), (H, W))

    return out, x1
